# baseline (speedup 1.0000x reference)
import jax
import jax.numpy as jnp
from jax.experimental import pallas as pl
from jax.experimental.pallas import tpu as pltpu

_N = 10000
_D = 128
_TJ = 2048
_TK = 2000


def _mm_kernel(e_ref, x_ref, o_ref):
    eb = e_ref[...].astype(jnp.bfloat16)
    xb = x_ref[...].astype(jnp.bfloat16)
    acc = jax.lax.dot_general(
        eb, xb, (((0,), (0,)), ((), ())),
        preferred_element_type=jnp.float32,
    )

    @pl.when(pl.program_id(1) == 0)
    def _init():
        o_ref[...] = acc

    @pl.when(pl.program_id(1) != 0)
    def _acc():
        o_ref[...] += acc


def kernel(x, a, e):
    x0 = x[0]
    return pl.pallas_call(
        _mm_kernel,
        grid=(pl.cdiv(_N, _TJ), pl.cdiv(_N, _TK)),
        in_specs=[
            pl.BlockSpec((_TK, _TJ), lambda j, k: (k, j)),
            pl.BlockSpec((_TK, _D), lambda j, k: (k, 0)),
        ],
        out_specs=pl.BlockSpec((_TJ, _D), lambda j, k: (j, 0)),
        out_shape=jax.ShapeDtypeStruct((_N, _D), jnp.float32),
        compiler_params=pltpu.CompilerParams(
            dimension_semantics=("parallel", "arbitrary"),
        ),
    )(e, x0)


# speedup vs baseline: 1.0609x; 1.0609x over previous
import jax
import jax.numpy as jnp
from jax.experimental import pallas as pl
from jax.experimental.pallas import tpu as pltpu

_N = 10000
_D = 128
_TJ = 512


def _mm_kernel(e_ref, x_ref, o_ref):
    eb = e_ref[...].astype(jnp.bfloat16)
    xb = x_ref[...].astype(jnp.bfloat16)
    o_ref[...] = jax.lax.dot_general(
        eb, xb, (((0,), (0,)), ((), ())),
        preferred_element_type=jnp.float32,
    )


def kernel(x, a, e):
    x0 = x[0]
    return pl.pallas_call(
        _mm_kernel,
        grid=(pl.cdiv(_N, _TJ),),
        in_specs=[
            pl.BlockSpec((_N, _TJ), lambda j: (0, j)),
            pl.BlockSpec((_N, _D), lambda j: (0, 0)),
        ],
        out_specs=pl.BlockSpec((_TJ, _D), lambda j: (j, 0)),
        out_shape=jax.ShapeDtypeStruct((_N, _D), jnp.float32),
        compiler_params=pltpu.CompilerParams(
            dimension_semantics=("parallel",),
        ),
    )(e, x0)
